# baseline (speedup 1.0000x reference)
import functools

import jax
import jax.numpy as jnp
from jax import lax
from jax.experimental import pallas as pl
from jax.experimental.pallas import tpu as pltpu
from jax.experimental.pallas import tpu_sc as plsc

N, D, B = 32768, 1024, 16
NC, NS = 2, 16
NW = NC * NS
RB = 32
LANES = 16
N_SC = 12288
CHUNK = N_SC // NW
NBLK = CHUNK // RB
RTC = 2048


def _sc_partials(x, batch_lengths):
    mesh = plsc.VectorSubcoreMesh(core_axis_name="c", subcore_axis_name="s")

    @functools.partial(
        pl.kernel,
        out_type=jax.ShapeDtypeStruct((NW * B, D), jnp.float32),
        mesh=mesh,
        scratch_types=[
            pltpu.VMEM((2, RB, D), jnp.float32),
            pltpu.VMEM((B, D), jnp.float32),
            pltpu.VMEM((LANES,), jnp.int32),
            pltpu.SMEM((B,), jnp.int32),
            pltpu.SMEM((B,), jnp.int32),
            pltpu.SemaphoreType.DMA,
        ],
        compiler_params=pltpu.CompilerParams(needs_layout_passes=False),
    )
    def k(x_hbm, len_hbm, out_hbm, buf, acc, len_v, lo_s, n_s, sem):
        c = lax.axis_index("c")
        s = lax.axis_index("s")
        w = s * NC + c
        base = w * CHUNK

        pltpu.sync_copy(len_hbm, len_v)

        lanes = lax.iota(jnp.int32, LANES)
        lens = len_v[...]
        ends_all = plsc.cumsum(lens)
        starts_all = ends_all - lens
        lo_all = jnp.maximum(starts_all, base)
        hi_all = jnp.minimum(ends_all, base + CHUNK)
        n_all = jnp.maximum(hi_all - lo_all, 0)

        def extract(j, _):
            lo_s[j] = jnp.sum(jnp.where(lanes == j, lo_all, 0)) - base
            n_s[j] = jnp.sum(jnp.where(lanes == j, n_all, 0))
            return 0

        lax.fori_loop(0, B, extract, 0)

        zeros = jnp.zeros((LANES,), jnp.float32)

        def block_copy(blk, slot):
            return pltpu.make_async_copy(
                x_hbm.at[pl.ds(base + blk * RB, RB), :],
                buf.at[slot],
                sem,
            )

        UNROLL = 16

        def run_j(j, _):
            lo_j = lo_s[j]
            n_j = n_s[j]

            @pl.when(n_j > 0)
            def _():
                blo = lax.div(lo_j, RB)
                bhi = lax.div(lo_j + n_j - 1, RB)
                block_copy(blo, 0).start()

                def blk_body(blk, slot):
                    block_copy(blk, slot).wait()

                    @pl.when(blk < bhi)
                    def _():
                        block_copy(blk + 1, 1 - slot).start()

                    wlo = blk * RB
                    lo_w = jnp.maximum(lo_j, wlo)
                    hi_w = jnp.minimum(lo_j + n_j, wlo + RB)
                    n_w = hi_w - lo_w
                    r0 = lo_w - wlo
                    nu = n_w - lax.rem(n_w, UNROLL)

                    def make_cs_body(first):
                        def cs_body(cs, __):
                            c0 = cs * (2 * LANES)

                            def body_u(kk, carry):
                                a0, a1 = carry
                                r = r0 + kk * UNROLL
                                for t in range(UNROLL):
                                    a0 = a0 + buf[
                                        slot, r + t, pl.ds(c0, LANES)
                                    ]
                                    a1 = a1 + buf[
                                        slot, r + t, pl.ds(c0 + LANES, LANES)
                                    ]
                                return (a0, a1)

                            a0, a1 = lax.fori_loop(
                                0, nu // UNROLL, body_u, (zeros, zeros)
                            )

                            def body_rem(i, carry):
                                a0, a1 = carry
                                a0 = a0 + buf[slot, r0 + i, pl.ds(c0, LANES)]
                                a1 = a1 + buf[
                                    slot, r0 + i, pl.ds(c0 + LANES, LANES)
                                ]
                                return (a0, a1)

                            a0, a1 = lax.fori_loop(nu, n_w, body_rem, (a0, a1))
                            if first:
                                acc[j, pl.ds(c0, LANES)] = a0
                                acc[j, pl.ds(c0 + LANES, LANES)] = a1
                            else:
                                plsc.addupdate(acc.at[j, pl.ds(c0, LANES)], a0)
                                plsc.addupdate(
                                    acc.at[j, pl.ds(c0 + LANES, LANES)], a1
                                )
                            return 0

                        return cs_body

                    def make_cs_full(first):
                        def cs_full(cs, __):
                            c0 = cs * (2 * LANES)
                            a0 = buf[slot, 0, pl.ds(c0, LANES)]
                            a1 = buf[slot, 0, pl.ds(c0 + LANES, LANES)]
                            b0 = buf[slot, 1, pl.ds(c0, LANES)]
                            b1 = buf[slot, 1, pl.ds(c0 + LANES, LANES)]
                            for t in range(2, RB, 2):
                                a0 = a0 + buf[slot, t, pl.ds(c0, LANES)]
                                a1 = a1 + buf[slot, t, pl.ds(c0 + LANES, LANES)]
                                b0 = b0 + buf[slot, t + 1, pl.ds(c0, LANES)]
                                b1 = b1 + buf[
                                    slot, t + 1, pl.ds(c0 + LANES, LANES)
                                ]
                            a0 = a0 + b0
                            a1 = a1 + b1
                            if first:
                                acc[j, pl.ds(c0, LANES)] = a0
                                acc[j, pl.ds(c0 + LANES, LANES)] = a1
                            else:
                                plsc.addupdate(acc.at[j, pl.ds(c0, LANES)], a0)
                                plsc.addupdate(
                                    acc.at[j, pl.ds(c0 + LANES, LANES)], a1
                                )
                            return 0

                        return cs_full

                    ncs = D // (2 * LANES)
                    full = n_w == RB
                    first = blk == blo

                    @pl.when(full & first)
                    def _():
                        lax.fori_loop(0, ncs, make_cs_full(True), 0)

                    @pl.when(full & jnp.logical_not(first))
                    def _():
                        lax.fori_loop(0, ncs, make_cs_full(False), 0)

                    @pl.when(jnp.logical_not(full) & first)
                    def _():
                        lax.fori_loop(0, ncs, make_cs_body(True), 0)

                    @pl.when(jnp.logical_not(full) & jnp.logical_not(first))
                    def _():
                        lax.fori_loop(0, ncs, make_cs_body(False), 0)

                    return 1 - slot

                lax.fori_loop(blo, bhi + 1, blk_body, 0)

            return 0

        lax.fori_loop(0, B, run_j, 0)

        def zero_absent(j, _):
            @pl.when(n_s[j] == 0)
            def _():
                def zc(cs, __):
                    acc[j, pl.ds(cs * LANES, LANES)] = zeros
                    return 0

                lax.fori_loop(0, D // LANES, zc, 0)

            return 0

        lax.fori_loop(0, B, zero_absent, 0)

        pltpu.sync_copy(acc, out_hbm.at[pl.ds(w * B, B), :])

    return k(x, batch_lengths)


def _tc_partial(x, starts_col, ends_col):
    def body(s_ref, e_ref, x_ref, o_ref):
        i = pl.program_id(0)
        rows = N_SC + i * RTC + lax.broadcasted_iota(jnp.int32, (B, RTC), 1)
        m = ((s_ref[...] <= rows) & (rows < e_ref[...])).astype(jnp.float32)
        part = jnp.dot(m, x_ref[...], preferred_element_type=jnp.float32)

        @pl.when(i == 0)
        def _():
            o_ref[...] = jnp.zeros_like(o_ref)

        o_ref[...] += part

    return pl.pallas_call(
        body,
        grid=((N - N_SC) // RTC,),
        in_specs=[
            pl.BlockSpec((B, 1), lambda i: (0, 0)),
            pl.BlockSpec((B, 1), lambda i: (0, 0)),
            pl.BlockSpec((RTC, D), lambda i: (N_SC // RTC + i, 0)),
        ],
        out_specs=pl.BlockSpec((B, D), lambda i: (0, 0)),
        out_shape=jax.ShapeDtypeStruct((B, D), jnp.float32),
    )(starts_col, ends_col, x)


def _combine(partials, tc_part, inv_len):
    def body(p_ref, t_ref, inv_ref, o_ref):
        o_ref[...] = (
            jnp.sum(p_ref[...].reshape(NW, B, D), axis=0) + t_ref[...]
        ) * inv_ref[...]

    return pl.pallas_call(
        body,
        out_shape=jax.ShapeDtypeStruct((B, D), jnp.float32),
    )(partials, tc_part, inv_len)


def kernel(x, batch_lengths):
    ends = jnp.cumsum(batch_lengths, dtype=jnp.int32)
    starts = ends - batch_lengths

    partials = _sc_partials(x, batch_lengths)
    tc_part = _tc_partial(x, starts[:, None], ends[:, None])
    inv_len = (1.0 / batch_lengths.astype(jnp.float32))[:, None]
    return _combine(partials, tc_part, inv_len)


# speedup vs baseline: 1.0131x; 1.0131x over previous
import functools

import jax
import jax.numpy as jnp
from jax import lax
from jax.experimental import pallas as pl
from jax.experimental.pallas import tpu as pltpu
from jax.experimental.pallas import tpu_sc as plsc

N, D, B = 32768, 1024, 16
NC, NS = 2, 16
NW = NC * NS
RB = 32
LANES = 16
N_SC = 12288
CHUNK = N_SC // NW
NBLK = CHUNK // RB
RTC = 2048


def _sc_partials(x, run_lo, run_n):
    mesh = plsc.VectorSubcoreMesh(core_axis_name="c", subcore_axis_name="s")

    @functools.partial(
        pl.kernel,
        out_type=jax.ShapeDtypeStruct((NW * B, D), jnp.float32),
        mesh=mesh,
        scratch_types=[
            pltpu.VMEM((2, RB, D), jnp.float32),
            pltpu.VMEM((B, D), jnp.float32),
            pltpu.VMEM((LANES,), jnp.int32),
            pltpu.VMEM((LANES,), jnp.int32),
            pltpu.SMEM((B,), jnp.int32),
            pltpu.SMEM((B,), jnp.int32),
            pltpu.SemaphoreType.DMA,
        ],
        compiler_params=pltpu.CompilerParams(needs_layout_passes=False),
    )
    def k(x_hbm, lo_hbm, n_hbm, out_hbm, buf, acc, lo_v, n_v, lo_s, n_s, sem):
        c = lax.axis_index("c")
        s = lax.axis_index("s")
        w = s * NC + c
        base = w * CHUNK

        pltpu.sync_copy(lo_hbm.at[pl.ds(w * B, B)], lo_v)
        pltpu.sync_copy(n_hbm.at[pl.ds(w * B, B)], n_v)

        lanes = lax.iota(jnp.int32, LANES)
        lo_all = lo_v[...]
        n_all = n_v[...]

        def extract(j, _):
            lo_s[j] = jnp.sum(jnp.where(lanes == j, lo_all, 0)) - base
            n_s[j] = jnp.sum(jnp.where(lanes == j, n_all, 0))
            return 0

        lax.fori_loop(0, B, extract, 0)

        zeros = jnp.zeros((LANES,), jnp.float32)

        def block_copy(blk, slot):
            return pltpu.make_async_copy(
                x_hbm.at[pl.ds(base + blk * RB, RB), :],
                buf.at[slot],
                sem,
            )

        UNROLL = 16

        def run_j(j, _):
            lo_j = lo_s[j]
            n_j = n_s[j]

            @pl.when(n_j > 0)
            def _():
                blo = lax.div(lo_j, RB)
                bhi = lax.div(lo_j + n_j - 1, RB)
                block_copy(blo, 0).start()

                def blk_body(blk, slot):
                    block_copy(blk, slot).wait()

                    @pl.when(blk < bhi)
                    def _():
                        block_copy(blk + 1, 1 - slot).start()

                    wlo = blk * RB
                    lo_w = jnp.maximum(lo_j, wlo)
                    hi_w = jnp.minimum(lo_j + n_j, wlo + RB)
                    n_w = hi_w - lo_w
                    r0 = lo_w - wlo
                    nu = n_w - lax.rem(n_w, UNROLL)

                    def make_cs_body(first):
                        def cs_body(cs, __):
                            c0 = cs * (2 * LANES)

                            def body_u(kk, carry):
                                a0, a1 = carry
                                r = r0 + kk * UNROLL
                                for t in range(UNROLL):
                                    a0 = a0 + buf[
                                        slot, r + t, pl.ds(c0, LANES)
                                    ]
                                    a1 = a1 + buf[
                                        slot, r + t, pl.ds(c0 + LANES, LANES)
                                    ]
                                return (a0, a1)

                            a0, a1 = lax.fori_loop(
                                0, nu // UNROLL, body_u, (zeros, zeros)
                            )

                            def body_rem(i, carry):
                                a0, a1 = carry
                                a0 = a0 + buf[slot, r0 + i, pl.ds(c0, LANES)]
                                a1 = a1 + buf[
                                    slot, r0 + i, pl.ds(c0 + LANES, LANES)
                                ]
                                return (a0, a1)

                            a0, a1 = lax.fori_loop(nu, n_w, body_rem, (a0, a1))
                            if first:
                                acc[j, pl.ds(c0, LANES)] = a0
                                acc[j, pl.ds(c0 + LANES, LANES)] = a1
                            else:
                                plsc.addupdate(acc.at[j, pl.ds(c0, LANES)], a0)
                                plsc.addupdate(
                                    acc.at[j, pl.ds(c0 + LANES, LANES)], a1
                                )
                            return 0

                        return cs_body

                    def make_cs_full(first):
                        def cs_full(cs, __):
                            c0 = cs * (2 * LANES)
                            a0 = buf[slot, 0, pl.ds(c0, LANES)]
                            a1 = buf[slot, 0, pl.ds(c0 + LANES, LANES)]
                            b0 = buf[slot, 1, pl.ds(c0, LANES)]
                            b1 = buf[slot, 1, pl.ds(c0 + LANES, LANES)]
                            for t in range(2, RB, 2):
                                a0 = a0 + buf[slot, t, pl.ds(c0, LANES)]
                                a1 = a1 + buf[slot, t, pl.ds(c0 + LANES, LANES)]
                                b0 = b0 + buf[slot, t + 1, pl.ds(c0, LANES)]
                                b1 = b1 + buf[
                                    slot, t + 1, pl.ds(c0 + LANES, LANES)
                                ]
                            a0 = a0 + b0
                            a1 = a1 + b1
                            if first:
                                acc[j, pl.ds(c0, LANES)] = a0
                                acc[j, pl.ds(c0 + LANES, LANES)] = a1
                            else:
                                plsc.addupdate(acc.at[j, pl.ds(c0, LANES)], a0)
                                plsc.addupdate(
                                    acc.at[j, pl.ds(c0 + LANES, LANES)], a1
                                )
                            return 0

                        return cs_full

                    ncs = D // (2 * LANES)
                    full = n_w == RB
                    first = blk == blo

                    @pl.when(full & first)
                    def _():
                        lax.fori_loop(0, ncs, make_cs_full(True), 0)

                    @pl.when(full & jnp.logical_not(first))
                    def _():
                        lax.fori_loop(0, ncs, make_cs_full(False), 0)

                    @pl.when(jnp.logical_not(full) & first)
                    def _():
                        lax.fori_loop(0, ncs, make_cs_body(True), 0)

                    @pl.when(jnp.logical_not(full) & jnp.logical_not(first))
                    def _():
                        lax.fori_loop(0, ncs, make_cs_body(False), 0)

                    return 1 - slot

                lax.fori_loop(blo, bhi + 1, blk_body, 0)

            return 0

        lax.fori_loop(0, B, run_j, 0)

        def zero_absent(j, _):
            @pl.when(n_s[j] == 0)
            def _():
                def zc(cs, __):
                    acc[j, pl.ds(cs * LANES, LANES)] = zeros
                    return 0

                lax.fori_loop(0, D // LANES, zc, 0)

            return 0

        lax.fori_loop(0, B, zero_absent, 0)

        pltpu.sync_copy(acc, out_hbm.at[pl.ds(w * B, B), :])

    return k(x, run_lo, run_n)


def _tc_partial(x, starts_col, ends_col):
    def body(s_ref, e_ref, x_ref, o_ref):
        i = pl.program_id(0)
        rows = N_SC + i * RTC + lax.broadcasted_iota(jnp.int32, (B, RTC), 1)
        m = ((s_ref[...] <= rows) & (rows < e_ref[...])).astype(jnp.float32)
        part = jnp.dot(m, x_ref[...], preferred_element_type=jnp.float32)

        @pl.when(i == 0)
        def _():
            o_ref[...] = jnp.zeros_like(o_ref)

        o_ref[...] += part

    return pl.pallas_call(
        body,
        grid=((N - N_SC) // RTC,),
        in_specs=[
            pl.BlockSpec((B, 1), lambda i: (0, 0)),
            pl.BlockSpec((B, 1), lambda i: (0, 0)),
            pl.BlockSpec((RTC, D), lambda i: (N_SC // RTC + i, 0)),
        ],
        out_specs=pl.BlockSpec((B, D), lambda i: (0, 0)),
        out_shape=jax.ShapeDtypeStruct((B, D), jnp.float32),
    )(starts_col, ends_col, x)


def _combine(partials, tc_part, inv_len):
    def body(p_ref, t_ref, inv_ref, o_ref):
        o_ref[...] = (
            jnp.sum(p_ref[...].reshape(NW, B, D), axis=0) + t_ref[...]
        ) * inv_ref[...]

    return pl.pallas_call(
        body,
        out_shape=jax.ShapeDtypeStruct((B, D), jnp.float32),
    )(partials, tc_part, inv_len)


def kernel(x, batch_lengths):
    ends = jnp.cumsum(batch_lengths, dtype=jnp.int32)
    starts = ends - batch_lengths

    wlo = jnp.arange(NW, dtype=jnp.int32)[:, None] * CHUNK
    whi = wlo + CHUNK
    lo = jnp.maximum(starts[None, :], wlo)
    hi = jnp.minimum(ends[None, :], whi)
    n = jnp.maximum(hi - lo, 0)

    partials = _sc_partials(x, lo.reshape(-1), n.reshape(-1))
    tc_part = _tc_partial(x, starts[:, None], ends[:, None])
    inv_len = (1.0 / batch_lengths.astype(jnp.float32))[:, None]
    return _combine(partials, tc_part, inv_len)
